# baseline (speedup 1.0000x reference)
import functools

import jax
import jax.numpy as jnp
from jax import lax
from jax.experimental import pallas as pl
from jax.experimental.pallas import tpu as pltpu
from jax.experimental.pallas import tpu_sc as plsc

NUM_CORES = 2
NUM_SUBCORES = 16
LANES = 16
NW = NUM_CORES * NUM_SUBCORES

BATCH = 16384
EMBED_K = 64
BPW = BATCH // NW
CHUNK = 256


def _sc_body(users_h, items_h, gu2_h, gi2_h, xui_h, gamma_u_h, gamma_i_h,
             idx_u, idx_i, row_u, row_i, raw_u, raw_i, out_u, out_i, xui_v,
             sem_u, sem_i, sem_ou, sem_oi):
    wid = lax.axis_index("s") * NUM_CORES + lax.axis_index("c")
    base = wid * BPW

    pltpu.sync_copy(users_h.at[pl.ds(base, BPW)], idx_u)
    pltpu.sync_copy(items_h.at[pl.ds(base, BPW)], idx_i)

    def shift(g, _):
        sl = pl.ds(g * LANES, LANES)
        row_u[sl] = lax.shift_right_logical(idx_u[sl], 1)
        row_i[sl] = lax.shift_right_logical(idx_i[sl], 1)
        return _

    lax.fori_loop(0, BPW // LANES, shift, 0)

    lane = lax.iota(jnp.int32, LANES)

    for h in range(BPW // CHUNK):
        j0 = h * CHUNK
        cu = pltpu.async_copy(gu2_h.at[row_u.at[pl.ds(j0, CHUNK)]], raw_u,
                              sem_u)
        ci = pltpu.async_copy(gi2_h.at[row_i.at[pl.ds(j0, CHUNK)]], raw_i,
                              sem_i)
        cu.wait()
        ci.wait()

        def compact(g, _):
            u16 = idx_u[pl.ds(j0 + g * LANES, LANES)]
            i16 = idx_i[pl.ds(j0 + g * LANES, LANES)]
            for l in range(LANES):
                r = g * LANES + l
                uoff = (u16[l] & 1) * EMBED_K
                ioff = (i16[l] & 1) * EMBED_K
                for c in range(EMBED_K // LANES):
                    out_u[r, pl.ds(c * LANES, LANES)] = (
                        raw_u[r, pl.ds(uoff + c * LANES, LANES)])
                    out_i[r, pl.ds(c * LANES, LANES)] = (
                        raw_i[r, pl.ds(ioff + c * LANES, LANES)])
            return _

        lax.fori_loop(0, CHUNK // LANES, compact, 0)

        ou = pltpu.async_copy(out_u, gamma_u_h.at[pl.ds(base + j0, CHUNK)],
                              sem_ou)
        oi = pltpu.async_copy(out_i, gamma_i_h.at[pl.ds(base + j0, CHUNK)],
                              sem_oi)

        def dot(g, _):
            acc = jnp.zeros((LANES,), jnp.float32)
            row16 = g * LANES + lane
            col = jnp.zeros((LANES,), jnp.int32)
            for _k in range(EMBED_K):
                uu = plsc.load_gather(out_u, [row16, col])
                ii = plsc.load_gather(out_i, [row16, col])
                acc = acc + uu * ii
                col = col + 1
            xui_v[pl.ds(j0 + g * LANES, LANES)] = acc
            return _

        lax.fori_loop(0, CHUNK // LANES, dot, 0)

        ou.wait()
        oi.wait()

    pltpu.sync_copy(xui_v, xui_h.at[pl.ds(base, BPW)])


@jax.jit
def _rslogic2_sc(users, items, Gu2, Gi2):
    mesh = plsc.VectorSubcoreMesh(
        core_axis_name="c", subcore_axis_name="s",
        num_cores=NUM_CORES, num_subcores=NUM_SUBCORES)
    return pl.kernel(
        _sc_body,
        out_type=(
            jax.ShapeDtypeStruct((BATCH,), jnp.float32),
            jax.ShapeDtypeStruct((BATCH, EMBED_K), jnp.float32),
            jax.ShapeDtypeStruct((BATCH, EMBED_K), jnp.float32),
        ),
        mesh=mesh,
        compiler_params=pltpu.CompilerParams(
            needs_layout_passes=False, use_tc_tiling_on_sc=False),
        scratch_types=[
            pltpu.VMEM((BPW,), jnp.int32),
            pltpu.VMEM((BPW,), jnp.int32),
            pltpu.VMEM((BPW,), jnp.int32),
            pltpu.VMEM((BPW,), jnp.int32),
            pltpu.VMEM((CHUNK, 2 * EMBED_K), jnp.float32),
            pltpu.VMEM((CHUNK, 2 * EMBED_K), jnp.float32),
            pltpu.VMEM((CHUNK, EMBED_K), jnp.float32),
            pltpu.VMEM((CHUNK, EMBED_K), jnp.float32),
            pltpu.VMEM((BPW,), jnp.float32),
            pltpu.SemaphoreType.DMA,
            pltpu.SemaphoreType.DMA,
            pltpu.SemaphoreType.DMA,
            pltpu.SemaphoreType.DMA,
        ],
    )(users, items, Gu2, Gi2)


def kernel(users, items, Gu, Gi):
    Gu2 = Gu.reshape(Gu.shape[0] // 2, 2 * EMBED_K)
    Gi2 = Gi.reshape(Gi.shape[0] // 2, 2 * EMBED_K)
    xui, gamma_u, gamma_i = _rslogic2_sc(users, items, Gu2, Gi2)
    return (xui, gamma_u, gamma_i)


# speedup vs baseline: 1.0261x; 1.0261x over previous
import jax
import jax.numpy as jnp
from jax import lax
from jax.experimental import pallas as pl
from jax.experimental.pallas import tpu as pltpu
from jax.experimental.pallas import tpu_sc as plsc

NUM_CORES = 2
NUM_SUBCORES = 16
LANES = 16
NW = NUM_CORES * NUM_SUBCORES

BATCH = 16384
EMBED_K = 64
BPW = BATCH // NW


def _sc_body(users_h, items_h, gu_h, gi_h, xui_h, gamma_u_h, gamma_i_h,
             idx_u, idx_i, rows_u, rows_i, xui_v,
             sem_u, sem_i, sem_ou, sem_oi):
    wid = lax.axis_index("s") * NUM_CORES + lax.axis_index("c")
    base = wid * BPW

    pltpu.sync_copy(users_h.at[pl.ds(base, BPW)], idx_u)
    pltpu.sync_copy(items_h.at[pl.ds(base, BPW)], idx_i)

    cu = pltpu.async_copy(gu_h.at[idx_u], rows_u, sem_u)
    ci = pltpu.async_copy(gi_h.at[idx_i], rows_i, sem_i)
    cu.wait()
    ci.wait()

    ou = pltpu.async_copy(rows_u, gamma_u_h.at[pl.ds(base, BPW)], sem_ou)
    oi = pltpu.async_copy(rows_i, gamma_i_h.at[pl.ds(base, BPW)], sem_oi)

    lane = lax.iota(jnp.int32, LANES)

    def dot(g, _):
        acc = jnp.zeros((LANES,), jnp.float32)
        row16 = g * LANES + lane
        col = jnp.zeros((LANES,), jnp.int32)
        for _k in range(EMBED_K):
            uu = plsc.load_gather(rows_u, [row16, col])
            ii = plsc.load_gather(rows_i, [row16, col])
            acc = acc + uu * ii
            col = col + 1
        xui_v[pl.ds(g * LANES, LANES)] = acc
        return _

    lax.fori_loop(0, BPW // LANES, dot, 0)

    pltpu.sync_copy(xui_v, xui_h.at[pl.ds(base, BPW)])
    ou.wait()
    oi.wait()


@jax.jit
def _rslogic2_sc(users, items, Gu, Gi):
    mesh = plsc.VectorSubcoreMesh(
        core_axis_name="c", subcore_axis_name="s",
        num_cores=NUM_CORES, num_subcores=NUM_SUBCORES)
    return pl.kernel(
        _sc_body,
        out_type=(
            jax.ShapeDtypeStruct((BATCH,), jnp.float32),
            jax.ShapeDtypeStruct((BATCH, EMBED_K), jnp.float32),
            jax.ShapeDtypeStruct((BATCH, EMBED_K), jnp.float32),
        ),
        mesh=mesh,
        compiler_params=pltpu.CompilerParams(
            needs_layout_passes=False, use_tc_tiling_on_sc=False),
        scratch_types=[
            pltpu.VMEM((BPW,), jnp.int32),
            pltpu.VMEM((BPW,), jnp.int32),
            pltpu.VMEM((BPW, EMBED_K), jnp.float32),
            pltpu.VMEM((BPW, EMBED_K), jnp.float32),
            pltpu.VMEM((BPW,), jnp.float32),
            pltpu.SemaphoreType.DMA,
            pltpu.SemaphoreType.DMA,
            pltpu.SemaphoreType.DMA,
            pltpu.SemaphoreType.DMA,
        ],
    )(users, items, Gu, Gi)


def kernel(users, items, Gu, Gi):
    xui, gamma_u, gamma_i = _rslogic2_sc(users, items, Gu, Gi)
    return (xui, gamma_u, gamma_i)
